# baseline (speedup 1.0000x reference)
import jax
import jax.numpy as jnp
from jax import lax
from jax.experimental import pallas as pl
from jax.experimental.pallas import tpu as pltpu
from jax.experimental.pallas import tpu_sc as plsc

N_NODES = 10000
N_EDGES = 160000
DIM = 256
HALF = DIM // 2

NC = 2
NS = 16
L = 16

CHUNK = 32
NBUF = 3
CPT = 318
EPT = CPT * CHUNK
E_PAD = NS * EPT
PACK_SHIFT = 14

RCHUNK = CHUNK
NRCH = N_NODES // RCHUNK
RTAIL = N_NODES - NRCH * RCHUNK
RITERS = -(-NRCH // NS)


def _sc_body(x2_hbm, meta_hbm, val_hbm, w_hbm, out_hbm,
             acc_sh, w_v, meta_all, val_all, colv, rowv, rows, gsems, ssems):
    c = lax.axis_index("c")
    s = lax.axis_index("s")

    stage = rows[0]
    def zero_row(r, _):
        for d in range(HALF // L):
            stage[r, pl.ds(d * L, L)] = jnp.zeros((L,), jnp.float32)
        return 0
    lax.fori_loop(0, RCHUNK, zero_row, 0)
    for t in range(RITERS):
        k = s + NS * t
        @pl.when(k < NRCH)
        def _():
            r0 = pl.multiple_of(k * RCHUNK, RCHUNK)
            pltpu.sync_copy(stage, acc_sh.at[pl.ds(r0, RCHUNK)])
    @pl.when(s == 0)
    def _():
        pltpu.sync_copy(stage.at[pl.ds(0, RTAIL)],
                        acc_sh.at[pl.ds(NRCH * RCHUNK, RTAIL)])

    woff = pl.multiple_of(c * HALF, HALF)
    pltpu.sync_copy(w_hbm.at[pl.ds(woff, HALF)], w_v)
    ebase = pl.multiple_of(s * EPT, CHUNK)
    pltpu.sync_copy(meta_hbm.at[pl.ds(ebase, EPT)], meta_all)
    pltpu.sync_copy(val_hbm.at[pl.ds(ebase, EPT)], val_all)
    plsc.subcore_barrier()

    col_mask = jnp.full((L,), (1 << PACK_SHIFT) - 1, jnp.int32)

    def unpack_meta(j, b):
        for g in range(CHUNK // L):
            sl = pl.ds(pl.multiple_of(j * CHUNK, CHUNK) + g * L, L)
            p = meta_all[sl]
            dst = pl.ds(g * L, L)
            colv[b][dst] = ((p & col_mask) << 1) + c
            rowv[b][dst] = lax.shift_right_logical(p, PACK_SHIFT)

    def fire_gather(b):
        pltpu.async_copy(x2_hbm.at[colv[b]], rows[b], gsems[b])

    unpack_meta(0, 0)
    fire_gather(0)

    def chunk_body(j3, _):
        lane = lax.iota(jnp.int32, L)
        for b in range(NBUF):
            jb = j3 * NBUF + b
            bn = (b + 1) % NBUF
            pltpu.make_async_copy(x2_hbm.at[colv[b]], rows[b], gsems[b]).wait()
            @pl.when(jb >= 2)
            def _():
                pltpu.make_async_copy(rows[bn], acc_sh.at[rowv[bn]],
                                      ssems[bn]).wait()
            @pl.when(jb + 1 < CPT)
            def _():
                unpack_meta(jb + 1, bn)
                fire_gather(bn)
            def group_body(g, _):
                vv = val_all[pl.ds(pl.multiple_of(jb * CHUNK, CHUNK) + g * L, L)]
                for i in range(L):
                    sv = vv.at[lane * 0 + i].get(mode="promise_in_bounds")
                    e = g * L + i
                    for d in range(HALF // L):
                        sl = pl.ds(d * L, L)
                        rows[b][e, sl] = rows[b][e, sl] * sv
                return 0
            lax.fori_loop(0, CHUNK // L, group_body, 0)
            @pl.when(jb < CPT - 2)
            def _():
                pltpu.async_copy(rows[b], acc_sh.at[rowv[b]], ssems[b],
                                 add=True)
            @pl.when(jb >= CPT - 2)
            def _():
                pltpu.sync_copy(rows[b], acc_sh.at[rowv[b]], add=True)
        return 0
    lax.fori_loop(0, CPT // NBUF, chunk_body, 0)
    plsc.subcore_barrier()

    def relu_rows(n):
        def relu_row(r, _):
            for d in range(HALF // L):
                sl = pl.ds(d * L, L)
                stage[r, sl] = jnp.maximum(stage[r, sl] * w_v[sl], 0.0)
            return 0
        lax.fori_loop(0, n, relu_row, 0, unroll=2)

    for t in range(RITERS):
        k = s + NS * t
        @pl.when(k < NRCH)
        def _():
            r0 = pl.multiple_of(k * RCHUNK, RCHUNK)
            pltpu.sync_copy(acc_sh.at[pl.ds(r0, RCHUNK)], stage)
            relu_rows(RCHUNK)
            pltpu.sync_copy(stage,
                            out_hbm.at[pl.ds(r0, RCHUNK), pl.ds(woff, HALF)])
    @pl.when(s == 0)
    def _():
        r0 = NRCH * RCHUNK
        pltpu.sync_copy(acc_sh.at[pl.ds(r0, RTAIL)], stage.at[pl.ds(0, RTAIL)])
        relu_rows(RTAIL)
        pltpu.sync_copy(stage.at[pl.ds(0, RTAIL)],
                        out_hbm.at[pl.ds(r0, RTAIL), pl.ds(woff, HALF)])


def _make_kernel():
    mesh = plsc.VectorSubcoreMesh(core_axis_name="c", subcore_axis_name="s")

    def body(x2_hbm, meta_hbm, val_hbm, w_hbm, out_hbm, acc_sh, w_v,
             meta_all, val_all, *rest):
        colv = rest[0:NBUF]
        rowv = rest[NBUF:2 * NBUF]
        rows = rest[2 * NBUF:3 * NBUF]
        sems = rest[3 * NBUF:]
        _sc_body(x2_hbm, meta_hbm, val_hbm, w_hbm, out_hbm,
                 acc_sh, w_v, meta_all, val_all, colv, rowv, rows,
                 sems[0:NBUF], sems[NBUF:2 * NBUF])

    return pl.kernel(
        body,
        out_type=jax.ShapeDtypeStruct((N_NODES, DIM), jnp.float32),
        mesh=mesh,
        scratch_types=[
            pltpu.VMEM_SHARED((N_NODES, HALF), jnp.float32),
            pltpu.VMEM((HALF,), jnp.float32),
            pltpu.VMEM((EPT,), jnp.int32),
            pltpu.VMEM((EPT,), jnp.float32),
        ] + [pltpu.VMEM((CHUNK,), jnp.int32)] * NBUF
          + [pltpu.VMEM((CHUNK,), jnp.int32)] * NBUF
          + [pltpu.VMEM((CHUNK, HALF), jnp.float32)] * NBUF
          + [pltpu.SemaphoreType.DMA] * (2 * NBUF),
    )


@jax.jit
def kernel(x, edge_index, edge_vals, W):
    row = edge_index[0].astype(jnp.int32)
    col = edge_index[1].astype(jnp.int32)
    pad = E_PAD - N_EDGES
    meta = jnp.pad((row << PACK_SHIFT) | col, (0, pad))
    val1 = jnp.pad(edge_vals.astype(jnp.float32), (0, pad))
    x2 = x.reshape(2 * N_NODES, HALF)
    w1 = W.reshape(DIM)
    return _make_kernel()(x2, meta, val1, w1)


# speedup vs baseline: 1.3988x; 1.3988x over previous
import jax
import jax.numpy as jnp
from jax import lax
from jax.experimental import pallas as pl
from jax.experimental.pallas import tpu as pltpu
from jax.experimental.pallas import tpu_sc as plsc

N_NODES = 10000
N_EDGES = 160000
DIM = 256
HALF = DIM // 2

NC = 2
NS = 16
L = 16

CHUNK = 32
NBUF = 6
KAHEAD = 3
CPT = 318
EPT = CPT * CHUNK
E_PAD = NS * EPT
PACK_SHIFT = 14

RCHUNK = CHUNK
NRCH = N_NODES // RCHUNK
RTAIL = N_NODES - NRCH * RCHUNK
RITERS = -(-NRCH // NS)


def _sc_body(x2_hbm, meta_hbm, val_hbm, w_hbm, out_hbm,
             acc_sh, w_v, meta_all, val_all, colv, rowv, rows, gsems, ssems):
    c = lax.axis_index("c")
    s = lax.axis_index("s")

    stage = rows[0]
    def zero_row(r, _):
        for d in range(HALF // L):
            stage[r, pl.ds(d * L, L)] = jnp.zeros((L,), jnp.float32)
        return 0
    lax.fori_loop(0, RCHUNK, zero_row, 0)
    for t in range(RITERS):
        k = s + NS * t
        @pl.when(k < NRCH)
        def _():
            r0 = pl.multiple_of(k * RCHUNK, RCHUNK)
            pltpu.sync_copy(stage, acc_sh.at[pl.ds(r0, RCHUNK)])
    @pl.when(s == 0)
    def _():
        pltpu.sync_copy(stage.at[pl.ds(0, RTAIL)],
                        acc_sh.at[pl.ds(NRCH * RCHUNK, RTAIL)])

    woff = pl.multiple_of(c * HALF, HALF)
    pltpu.sync_copy(w_hbm.at[pl.ds(woff, HALF)], w_v)
    ebase = pl.multiple_of(s * EPT, CHUNK)
    pltpu.sync_copy(meta_hbm.at[pl.ds(ebase, EPT)], meta_all)
    pltpu.sync_copy(val_hbm.at[pl.ds(ebase, EPT)], val_all)
    plsc.subcore_barrier()

    col_mask = jnp.full((L,), (1 << PACK_SHIFT) - 1, jnp.int32)

    def unpack_meta(j, b):
        for g in range(CHUNK // L):
            sl = pl.ds(pl.multiple_of(j * CHUNK, CHUNK) + g * L, L)
            p = meta_all[sl]
            dst = pl.ds(g * L, L)
            colv[b][dst] = ((p & col_mask) << 1) + c
            rowv[b][dst] = lax.shift_right_logical(p, PACK_SHIFT)

    def fire_gather(b):
        pltpu.async_copy(x2_hbm.at[colv[b]], rows[b], gsems[b])

    for j0 in range(KAHEAD):
        unpack_meta(j0, j0)
        fire_gather(j0)

    def chunk_body(j6, _):
        lane = lax.iota(jnp.int32, L)
        for b in range(NBUF):
            jb = j6 * NBUF + b
            bk = (b + KAHEAD) % NBUF
            pltpu.make_async_copy(x2_hbm.at[colv[b]], rows[b], gsems[b]).wait()
            def group_body(g, _):
                vv = val_all[pl.ds(pl.multiple_of(jb * CHUNK, CHUNK) + g * L, L)]
                for i in range(L):
                    sv = vv.at[lane * 0 + i].get(mode="promise_in_bounds")
                    e = g * L + i
                    for d in range(HALF // L):
                        sl = pl.ds(d * L, L)
                        rows[b][e, sl] = rows[b][e, sl] * sv
                return 0
            lax.fori_loop(0, CHUNK // L, group_body, 0)
            @pl.when(jb < CPT - KAHEAD)
            def _():
                pltpu.async_copy(rows[b], acc_sh.at[rowv[b]], ssems[b],
                                 add=True)
            @pl.when(jb >= CPT - KAHEAD)
            def _():
                pltpu.sync_copy(rows[b], acc_sh.at[rowv[b]], add=True)
            @pl.when(jb >= KAHEAD)
            def _():
                pltpu.make_async_copy(rows[bk], acc_sh.at[rowv[bk]],
                                      ssems[bk]).wait()
            @pl.when(jb + KAHEAD < CPT)
            def _():
                unpack_meta(jb + KAHEAD, bk)
                fire_gather(bk)
        return 0
    lax.fori_loop(0, CPT // NBUF, chunk_body, 0)
    plsc.subcore_barrier()

    def relu_rows(n):
        def relu_row(r, _):
            for d in range(HALF // L):
                sl = pl.ds(d * L, L)
                stage[r, sl] = jnp.maximum(stage[r, sl] * w_v[sl], 0.0)
            return 0
        lax.fori_loop(0, n, relu_row, 0, unroll=2)

    for t in range(RITERS):
        k = s + NS * t
        @pl.when(k < NRCH)
        def _():
            r0 = pl.multiple_of(k * RCHUNK, RCHUNK)
            pltpu.sync_copy(acc_sh.at[pl.ds(r0, RCHUNK)], stage)
            relu_rows(RCHUNK)
            pltpu.sync_copy(stage,
                            out_hbm.at[pl.ds(r0, RCHUNK), pl.ds(woff, HALF)])
    @pl.when(s == 0)
    def _():
        r0 = NRCH * RCHUNK
        pltpu.sync_copy(acc_sh.at[pl.ds(r0, RTAIL)], stage.at[pl.ds(0, RTAIL)])
        relu_rows(RTAIL)
        pltpu.sync_copy(stage.at[pl.ds(0, RTAIL)],
                        out_hbm.at[pl.ds(r0, RTAIL), pl.ds(woff, HALF)])


def _make_kernel():
    mesh = plsc.VectorSubcoreMesh(core_axis_name="c", subcore_axis_name="s")

    def body(x2_hbm, meta_hbm, val_hbm, w_hbm, out_hbm, acc_sh, w_v,
             meta_all, val_all, *rest):
        colv = rest[0:NBUF]
        rowv = rest[NBUF:2 * NBUF]
        rows = rest[2 * NBUF:3 * NBUF]
        sems = rest[3 * NBUF:]
        _sc_body(x2_hbm, meta_hbm, val_hbm, w_hbm, out_hbm,
                 acc_sh, w_v, meta_all, val_all, colv, rowv, rows,
                 sems[0:NBUF], sems[NBUF:2 * NBUF])

    return pl.kernel(
        body,
        out_type=jax.ShapeDtypeStruct((N_NODES, DIM), jnp.float32),
        mesh=mesh,
        scratch_types=[
            pltpu.VMEM_SHARED((N_NODES, HALF), jnp.float32),
            pltpu.VMEM((HALF,), jnp.float32),
            pltpu.VMEM((EPT,), jnp.int32),
            pltpu.VMEM((EPT,), jnp.float32),
        ] + [pltpu.VMEM((CHUNK,), jnp.int32)] * NBUF
          + [pltpu.VMEM((CHUNK,), jnp.int32)] * NBUF
          + [pltpu.VMEM((CHUNK, HALF), jnp.float32)] * NBUF
          + [pltpu.SemaphoreType.DMA] * (2 * NBUF),
    )


@jax.jit
def kernel(x, edge_index, edge_vals, W):
    row = edge_index[0].astype(jnp.int32)
    col = edge_index[1].astype(jnp.int32)
    pad = E_PAD - N_EDGES
    meta = jnp.pad((row << PACK_SHIFT) | col, (0, pad))
    val1 = jnp.pad(edge_vals.astype(jnp.float32), (0, pad))
    x2 = x.reshape(2 * N_NODES, HALF)
    w1 = W.reshape(DIM)
    return _make_kernel()(x2, meta, val1, w1)
